# baseline (speedup 1.0000x reference)
import functools

import jax
import jax.numpy as jnp
from jax import lax
from jax.experimental import pallas as pl
from jax.experimental.pallas import tpu as pltpu
from jax.experimental.pallas import tpu_sc as plsc

_NC = 2
_NS = 16
_NW = _NC * _NS
_LANES = 16


def kernel(x, pe):
    B, S, D = x.shape
    R = 32
    SW = S // _NW
    n_sub = SW // R
    chunk = R * D
    n_vec = chunk // _LANES

    mesh = plsc.VectorSubcoreMesh(core_axis_name="c", subcore_axis_name="s")

    @functools.partial(
        pl.kernel,
        out_type=jax.ShapeDtypeStruct((B * S * D,), jnp.float32),
        mesh=mesh,
        scratch_types=[
            pltpu.VMEM((chunk,), jnp.float32),
            pltpu.VMEM((chunk,), jnp.float32),
        ],
    )
    def run(x_hbm, pe_hbm, out_hbm, pe_v, x_v):
        wid = lax.axis_index("s") * _NC + lax.axis_index("c")
        base = wid * SW

        def sub(t, carry):
            s0 = (base + t * R) * D
            pltpu.sync_copy(pe_hbm.at[pl.ds(s0, chunk)], pe_v)
            for b in range(B):
                xoff = b * S * D + s0
                pltpu.sync_copy(x_hbm.at[pl.ds(xoff, chunk)], x_v)

                def add_body(i, c2):
                    sl = pl.ds(i * _LANES, _LANES)
                    x_v[sl] = x_v[sl] + pe_v[sl]
                    return c2

                lax.fori_loop(0, n_vec, add_body, 0)
                pltpu.sync_copy(x_v, out_hbm.at[pl.ds(xoff, chunk)])
            return carry

        lax.fori_loop(0, n_sub, sub, 0)

    out = run(x.reshape(-1), pe.reshape(-1))
    return out.reshape(B, S, D)


# speedup vs baseline: 5.1413x; 5.1413x over previous
import functools

import jax
import jax.numpy as jnp
from jax import lax
from jax.experimental import pallas as pl
from jax.experimental.pallas import tpu as pltpu
from jax.experimental.pallas import tpu_sc as plsc

_NC = 2
_NS = 16
_NW = _NC * _NS
_LANES = 16


def kernel(x, pe):
    B, S, D = x.shape
    R = 16
    SW = S // _NW
    n_sub = SW // R
    n_vec = R * D // _LANES
    n_col = D // _LANES
    J = n_sub * B

    mesh = plsc.VectorSubcoreMesh(core_axis_name="c", subcore_axis_name="s")

    @functools.partial(
        pl.kernel,
        out_type=jax.ShapeDtypeStruct((B, S, D), jnp.float32),
        mesh=mesh,
        scratch_types=[
            pltpu.VMEM((R, D), jnp.float32),
            pltpu.VMEM((R, D), jnp.float32),
            pltpu.VMEM((R, D), jnp.float32),
            pltpu.VMEM((R, D), jnp.float32),
            pltpu.VMEM((R, D), jnp.float32),
            pltpu.SemaphoreType.DMA,
            pltpu.SemaphoreType.DMA,
            pltpu.SemaphoreType.DMA,
            pltpu.SemaphoreType.DMA,
            pltpu.SemaphoreType.DMA,
            pltpu.SemaphoreType.DMA,
            pltpu.SemaphoreType.DMA,
            pltpu.SemaphoreType.DMA,
        ],
    )
    def run(x_hbm, pe_hbm, out_hbm, xb0, xb1, xb2, pb0, pb1,
            xs0, xs1, xs2, os0, os1, os2, ps0, ps1):
        xb = [xb0, xb1, xb2]
        peb = [pb0, pb1]
        xsem = [xs0, xs1, xs2]
        osem = [os0, os1, os2]
        psem = [ps0, ps1]

        wid = lax.axis_index("s") * _NC + lax.axis_index("c")
        base = wid * SW

        def pe_block(t):
            return pe_hbm.at[pl.ds(base + t * R, R)]

        def x_block(j):
            t, b = divmod(j, B)
            return x_hbm.at[b].at[pl.ds(base + t * R, R)]

        def out_block(j):
            t, b = divmod(j, B)
            return out_hbm.at[b].at[pl.ds(base + t * R, R)]

        pe_cp = [None] * n_sub
        x_cp = [None] * J
        o_cp = [None] * J
        pe_cp[0] = pltpu.async_copy(pe_block(0), peb[0], psem[0])
        x_cp[0] = pltpu.async_copy(x_block(0), xb[0], xsem[0])

        for j in range(J):
            t, b = divmod(j, B)
            slot = j % 3
            if b == 0 and t + 1 < n_sub:
                nt = t + 1
                pe_cp[nt] = pltpu.async_copy(
                    pe_block(nt), peb[nt % 2], psem[nt % 2])
            if j + 1 < J:
                nslot = (j + 1) % 3
                if j - 2 >= 0:
                    o_cp[j - 2].wait()
                x_cp[j + 1] = pltpu.async_copy(
                    x_block(j + 1), xb[nslot], xsem[nslot])
            if b == 0:
                pe_cp[t].wait()
            x_cp[j].wait()

            xr = xb[slot]
            pr = peb[t % 2]

            @plsc.parallel_loop(0, n_vec, step=1, unroll=8)
            def add_body(i):
                r = i // n_col
                c = (i % n_col) * _LANES
                sl = pl.ds(c, _LANES)
                plsc.addupdate(xr.at[r, sl], pr[r, sl])

            o_cp[j] = pltpu.async_copy(xr, out_block(j), osem[slot])

        o_cp[J - 2].wait()
        o_cp[J - 1].wait()

    return run(x, pe)


# speedup vs baseline: 5.4563x; 1.0613x over previous
import functools

import jax
import jax.numpy as jnp
from jax import lax
from jax.experimental import pallas as pl
from jax.experimental.pallas import tpu as pltpu
from jax.experimental.pallas import tpu_sc as plsc

_NC = 2
_NS = 16
_NW = _NC * _NS
_LANES = 16
_NBUF = 5
_AHEAD = 3


def kernel(x, pe):
    B, S, D = x.shape
    R = 16
    SW = S // _NW
    n_sub = SW // R
    n_vec = R * D // _LANES
    n_col = D // _LANES
    J = n_sub * B

    mesh = plsc.VectorSubcoreMesh(core_axis_name="c", subcore_axis_name="s")

    scratch = (
        [pltpu.VMEM((R, D), jnp.float32) for _ in range(_NBUF)]
        + [pltpu.VMEM((R, D), jnp.float32) for _ in range(2)]
        + [pltpu.SemaphoreType.DMA for _ in range(_NBUF)]
        + [pltpu.SemaphoreType.DMA for _ in range(_NBUF)]
        + [pltpu.SemaphoreType.DMA for _ in range(2)]
    )

    @functools.partial(
        pl.kernel,
        out_type=jax.ShapeDtypeStruct((B, S, D), jnp.float32),
        mesh=mesh,
        scratch_types=scratch,
    )
    def run(x_hbm, pe_hbm, out_hbm, *bufs):
        xb = list(bufs[0:_NBUF])
        peb = list(bufs[_NBUF:_NBUF + 2])
        xsem = list(bufs[_NBUF + 2:2 * _NBUF + 2])
        osem = list(bufs[2 * _NBUF + 2:3 * _NBUF + 2])
        psem = list(bufs[3 * _NBUF + 2:3 * _NBUF + 4])

        wid = lax.axis_index("s") * _NC + lax.axis_index("c")
        base = wid * SW

        def pe_block(t):
            return pe_hbm.at[pl.ds(base + t * R, R)]

        def x_block(j):
            t, b = divmod(j, B)
            return x_hbm.at[b].at[pl.ds(base + t * R, R)]

        def out_block(j):
            t, b = divmod(j, B)
            return out_hbm.at[b].at[pl.ds(base + t * R, R)]

        pe_cp = [None] * n_sub
        x_cp = [None] * J
        o_cp = [None] * J
        waited = [False] * J
        pe_cp[0] = pltpu.async_copy(pe_block(0), peb[0], psem[0])
        for j in range(min(_AHEAD, J)):
            x_cp[j] = pltpu.async_copy(x_block(j), xb[j % _NBUF], xsem[j % _NBUF])

        for j in range(J):
            t, b = divmod(j, B)
            slot = j % _NBUF
            if b == 0 and t + 1 < n_sub:
                nt = t + 1
                pe_cp[nt] = pltpu.async_copy(
                    pe_block(nt), peb[nt % 2], psem[nt % 2])
            nj = j + _AHEAD
            if nj < J:
                prev = nj - _NBUF
                if prev >= 0:
                    o_cp[prev].wait()
                    waited[prev] = True
                x_cp[nj] = pltpu.async_copy(
                    x_block(nj), xb[nj % _NBUF], xsem[nj % _NBUF])
            if b == 0:
                pe_cp[t].wait()
            x_cp[j].wait()

            xr = xb[slot]
            pr = peb[t % 2]

            @plsc.parallel_loop(0, n_vec, step=1, unroll=8)
            def add_body(i):
                r = i // n_col
                c = (i % n_col) * _LANES
                sl = pl.ds(c, _LANES)
                plsc.addupdate(xr.at[r, sl], pr[r, sl])

            o_cp[j] = pltpu.async_copy(xr, out_block(j), osem[slot])

        for j in range(J):
            if not waited[j]:
                o_cp[j].wait()

    return run(x, pe)
